# baseline (speedup 1.0000x reference)
import functools

import jax
import jax.numpy as jnp
from jax import lax
from jax.experimental import pallas as pl
from jax.experimental.pallas import tpu as pltpu
from jax.experimental.pallas import tpu_sc as plsc

MB = 128
TOPK = 2


def _router_body(nblk, x_ref, rw_ref, d0_ref, d1_ref, w0_ref, w1_ref,
                 eob_ref, src_ref, nv_ref):
    xb = x_ref[...]
    t, _ = xb.shape
    e = rw_ref.shape[1]
    logits = jnp.dot(xb, rw_ref[...], preferred_element_type=jnp.float32)
    m = jnp.max(logits, axis=1, keepdims=True)
    p = jnp.exp(logits - m)
    p = p / jnp.sum(p, axis=1, keepdims=True)
    ii = lax.broadcasted_iota(jnp.int32, (t, e), 1)
    p1 = jnp.max(p, axis=1, keepdims=True)
    i1 = jnp.min(jnp.where(p == p1, ii, e), axis=1, keepdims=True)
    m1 = ii == i1
    pm = jnp.where(m1, -jnp.inf, p)
    p2 = jnp.max(pm, axis=1, keepdims=True)
    i2 = jnp.min(jnp.where(pm == p2, ii, e), axis=1, keepdims=True)
    m2 = ii == i2
    s = p1 + p2
    w0_ref[...] = (p1 / s).reshape(1, t)
    w1_ref[...] = (p2 / s).reshape(1, t)

    f1 = m1.astype(jnp.float32)
    f2 = m2.astype(jnp.float32)

    cb = 256
    ri = lax.broadcasted_iota(jnp.int32, (cb, cb), 0)
    ci = lax.broadcasted_iota(jnp.int32, (cb, cb), 1)
    tri = (ri >= ci).astype(jnp.float32)

    def cum(mat):
        chunks = []
        carry = jnp.zeros((1, e), jnp.float32)
        for c in range(t // cb):
            blk = mat[c * cb:(c + 1) * cb, :]
            cbk = jnp.dot(tri, blk, preferred_element_type=jnp.float32) + carry
            carry = cbk[cb - 1:cb, :]
            chunks.append(cbk)
        return jnp.concatenate(chunks, 0), carry

    c1, cnt1 = cum(f1)
    c2, cnt2 = cum(f2)
    cnt = cnt1 + cnt2
    nb = jnp.ceil(cnt * (1.0 / MB))
    eri = lax.broadcasted_iota(jnp.int32, (e, e), 0)
    eci = lax.broadcasted_iota(jnp.int32, (e, e), 1)
    triu = (eri < eci).astype(jnp.float32)
    pado = jnp.dot(nb, triu, preferred_element_type=jnp.float32) * MB
    pado_b = jnp.broadcast_to(pado, (t, e))
    cnt1_b = jnp.broadcast_to(cnt1, (t, e))
    d0 = jnp.sum(jnp.where(m1, pado_b + c1, 0.0), axis=1) - 1.0
    d1 = jnp.sum(jnp.where(m2, pado_b + cnt1_b + c2, 0.0), axis=1) - 1.0
    d0_ref[...] = d0.astype(jnp.int32).reshape(1, t)
    d1_ref[...] = d1.astype(jnp.int32).reshape(1, t)

    cume = jnp.dot(nb, triu,
                   preferred_element_type=jnp.float32).astype(jnp.int32)
    total = jnp.sum(nb).astype(jnp.int32)
    jb = lax.broadcasted_iota(jnp.int32, (nblk, e), 0)
    eob = jnp.sum((jb >= jnp.broadcast_to(cume, (nblk, e))).astype(
        jnp.int32), axis=1) - 1
    jv = lax.broadcasted_iota(jnp.int32, (nblk, 1), 0).reshape(nblk)
    lastv = jnp.maximum(total - 1, 0)
    srcv = jnp.where(jv < total, jv, lastv)
    jj = lax.broadcasted_iota(jnp.int32, (nblk, nblk), 1)
    sel = jj == srcv[:, None]
    eobp = jnp.sum(
        jnp.where(sel, jnp.broadcast_to(eob[None, :], (nblk, nblk)), 0),
        axis=1)
    eob_ref[...] = eobp.reshape(1, nblk)
    src_ref[...] = srcv.reshape(1, nblk)
    nv_ref[...] = total.reshape(1, 1)


@functools.partial(jax.jit, static_argnames=("nblk",))
def _router_call(x, rw, nblk):
    t, d = x.shape
    e = rw.shape[1]
    return pl.pallas_call(
        functools.partial(_router_body, nblk),
        grid=(1,),
        in_specs=[
            pl.BlockSpec((t, d), lambda i: (0, 0)),
            pl.BlockSpec((d, e), lambda i: (0, 0)),
        ],
        out_specs=[
            pl.BlockSpec((1, t), lambda i: (0, 0)),
            pl.BlockSpec((1, t), lambda i: (0, 0)),
            pl.BlockSpec((1, t), lambda i: (0, 0)),
            pl.BlockSpec((1, t), lambda i: (0, 0)),
            pl.BlockSpec((1, nblk), lambda i: (0, 0)),
            pl.BlockSpec((1, nblk), lambda i: (0, 0)),
            pl.BlockSpec((1, 1), lambda i: (0, 0)),
        ],
        out_shape=[
            jax.ShapeDtypeStruct((1, t), jnp.int32),
            jax.ShapeDtypeStruct((1, t), jnp.int32),
            jax.ShapeDtypeStruct((1, t), jnp.float32),
            jax.ShapeDtypeStruct((1, t), jnp.float32),
            jax.ShapeDtypeStruct((1, nblk), jnp.int32),
            jax.ShapeDtypeStruct((1, nblk), jnp.int32),
            jax.ShapeDtypeStruct((1, 1), jnp.int32),
        ],
    )(x, rw)


def _make_dispatch(t, d, s, nw):
    sw = s // nw
    ng = sw // 32
    mesh = plsc.VectorSubcoreMesh(core_axis_name="c", subcore_axis_name="s")

    @functools.partial(
        pl.kernel,
        mesh=mesh,
        compiler_params=pltpu.CompilerParams(needs_layout_passes=False),
        out_type=[
            jax.ShapeDtypeStruct((s, d), jnp.float32),
            jax.ShapeDtypeStruct((s,), jnp.float32),
        ],
        scratch_types=[
            pltpu.VMEM((t,), jnp.int32),
            pltpu.VMEM((t,), jnp.int32),
            pltpu.VMEM((t,), jnp.float32),
            pltpu.VMEM((t,), jnp.float32),
            pltpu.VMEM((sw,), jnp.int32),
            pltpu.VMEM((sw,), jnp.float32),
        ] + [pltpu.VMEM((16, d), jnp.float32) for _ in range(6)]
          + [pltpu.SemaphoreType.DMA for _ in range(12)],
    )
    def dispatch(d0_hbm, d1_hbm, w0_hbm, w1_hbm, x_hbm, xs_hbm, ws_hbm,
                 d0_v, d1_v, w0_v, w1_v, inv1, wv,
                 b0, b1, b2, b3, b4, b5,
                 g0, g1, g2, g3, g4, g5,
                 h0, h1, h2, h3, h4, h5):
        wid = lax.axis_index("s") * 2 + lax.axis_index("c")
        lo = wid * sw

        zi = jnp.zeros((16,), jnp.int32)
        zf = jnp.zeros((16,), jnp.float32)
        for c in range(sw // 16):
            inv1[pl.ds(c * 16, 16)] = zi
            wv[pl.ds(c * 16, 16)] = zf

        pltpu.sync_copy(d0_hbm, d0_v)
        pltpu.sync_copy(d1_hbm, d1_v)
        pltpu.sync_copy(w0_hbm, w0_v)
        pltpu.sync_copy(w1_hbm, w1_v)

        iota16 = lax.iota(jnp.int32, 16)

        def scan_pass(dv, wsrc):
            def body(c, carry):
                off = c * 16
                dd = dv[pl.ds(off, 16)]
                rel = dd - lo
                msk = (rel >= 0) & (rel < sw)
                relc = jnp.where(msk, rel, 0)
                toks = iota16 + off
                plsc.store_scatter(inv1, [relc], toks, mask=msk)
                plsc.store_scatter(wv, [relc], wsrc[pl.ds(off, 16)], mask=msk)
                return carry
            lax.fori_loop(0, t // 16, body, 0)

        scan_pass(d0_v, w0_v)
        scan_pass(d1_v, w1_v)

        pltpu.sync_copy(wv, ws_hbm.at[pl.ds(lo, sw)])

        nb = 6
        bufs = [b0, b1, b2, b3, b4, b5]
        gsem = [g0, g1, g2, g3, g4, g5]
        wsem = [h0, h1, h2, h3, h4, h5]
        nch = sw // 16
        gds = [None] * nch
        wds = [None] * nch

        def fire(j):
            gds[j] = pltpu.async_copy(
                x_hbm.at[inv1.at[pl.ds(j * 16, 16)]], bufs[j % nb],
                gsem[j % nb])

        for j in range(min(nb, nch)):
            fire(j)
        for j in range(nch):
            gds[j].wait()
            wds[j] = pltpu.async_copy(
                bufs[j % nb], xs_hbm.at[pl.ds(lo + j * 16, 16)],
                wsem[j % nb])
            if j + nb < nch:
                wds[j].wait()
                fire(j + nb)
        for j in range(max(0, nch - nb), nch):
            wds[j].wait()

    return dispatch


def _gmm_body(eob_ref, src_ref, nv_ref, xs_ref, ws_ref, wg_ref, wu_ref,
              wd_ref, ys_ref, wg16, wu16, wd16):
    b = pl.program_id(0)
    valid = b < nv_ref[0]
    changed = (b == 0) | (eob_ref[b] != eob_ref[jnp.maximum(b - 1, 0)])

    @pl.when(valid & changed)
    def _cast():
        wg16[...] = wg_ref[0].astype(jnp.bfloat16)
        wu16[...] = wu_ref[0].astype(jnp.bfloat16)
        wd16[...] = wd_ref[0].astype(jnp.bfloat16)

    @pl.when(valid)
    def _compute():
        x16 = xs_ref[...].astype(jnp.bfloat16)
        g = jnp.dot(x16, wg16[...], preferred_element_type=jnp.float32)
        u = jnp.dot(x16, wu16[...], preferred_element_type=jnp.float32)
        h = (g * jax.nn.sigmoid(g)) * u
        y = jnp.dot(h.astype(jnp.bfloat16), wd16[...],
                    preferred_element_type=jnp.float32)
        ys_ref[...] = y * ws_ref[0, 0, :][:, None]


@jax.jit
def _gmm_call(eobp, src, nv, xs, ws3, wg, wu, wd):
    s, d = xs.shape
    e, _, f = wg.shape
    nblk = ws3.shape[0]
    grid_spec = pltpu.PrefetchScalarGridSpec(
        num_scalar_prefetch=3,
        grid=(nblk,),
        in_specs=[
            pl.BlockSpec((MB, d), lambda b, eo, sr, nv_: (sr[b], 0)),
            pl.BlockSpec((1, 1, MB), lambda b, eo, sr, nv_: (sr[b], 0, 0)),
            pl.BlockSpec((1, d, f), lambda b, eo, sr, nv_: (eo[b], 0, 0)),
            pl.BlockSpec((1, d, f), lambda b, eo, sr, nv_: (eo[b], 0, 0)),
            pl.BlockSpec((1, f, d), lambda b, eo, sr, nv_: (eo[b], 0, 0)),
        ],
        out_specs=pl.BlockSpec((MB, d), lambda b, eo, sr, nv_: (sr[b], 0)),
        scratch_shapes=[
            pltpu.VMEM((d, f), jnp.bfloat16),
            pltpu.VMEM((d, f), jnp.bfloat16),
            pltpu.VMEM((f, d), jnp.bfloat16),
        ],
    )
    return pl.pallas_call(
        _gmm_body,
        grid_spec=grid_spec,
        out_shape=jax.ShapeDtypeStruct((s, d), jnp.float32),
    )(eobp, src, nv, xs, ws3, wg, wu, wd)


def _make_combine(t, d, s, nw):
    tw = t // nw
    nc = tw // 16
    mesh = plsc.VectorSubcoreMesh(core_axis_name="c", subcore_axis_name="s")

    @functools.partial(
        pl.kernel,
        mesh=mesh,
        compiler_params=pltpu.CompilerParams(needs_layout_passes=False),
        out_type=jax.ShapeDtypeStruct((t, d), jnp.float32),
        scratch_types=[
            pltpu.VMEM((tw,), jnp.int32),
            pltpu.VMEM((tw,), jnp.int32),
            pltpu.VMEM((16, d), jnp.float32),
            pltpu.VMEM((16, d), jnp.float32),
            pltpu.VMEM((16, d), jnp.float32),
            pltpu.VMEM((16, d), jnp.float32),
            pltpu.SemaphoreType.DMA,
            pltpu.SemaphoreType.DMA,
            pltpu.SemaphoreType.DMA,
            pltpu.SemaphoreType.DMA,
            pltpu.SemaphoreType.DMA,
            pltpu.SemaphoreType.DMA,
        ],
    )
    def combine(d0_hbm, d1_hbm, ys_hbm, out_hbm,
                d0t, d1t, a0, b0, a1, b1,
                sa0, sb0, sa1, sb1, sw0, sw1):
        wid = lax.axis_index("s") * 2 + lax.axis_index("c")
        lo = wid * tw
        pltpu.sync_copy(d0_hbm.at[pl.ds(lo, tw)], d0t)
        pltpu.sync_copy(d1_hbm.at[pl.ds(lo, tw)], d1t)

        abufs = [a0, a1]
        bbufs = [b0, b1]
        asem = [sa0, sa1]
        bsem = [sb0, sb1]
        wsem = [sw0, sw1]
        ga = [None] * nc
        gb = [None] * nc
        wd_ = [None] * nc

        def start(j):
            sl = pl.ds(j * 16, 16)
            ga[j] = pltpu.async_copy(ys_hbm.at[d0t.at[sl]], abufs[j % 2],
                                     asem[j % 2])
            gb[j] = pltpu.async_copy(ys_hbm.at[d1t.at[sl]], bbufs[j % 2],
                                     bsem[j % 2])

        start(0)
        for j in range(nc):
            ga[j].wait()
            gb[j].wait()
            if j + 1 < nc:
                if j - 1 >= 0:
                    wd_[j - 1].wait()
                start(j + 1)
            a = abufs[j % 2]
            b = bbufs[j % 2]

            def row_body(r, carry):
                for c in range(d // 16):
                    sl = pl.ds(c * 16, 16)
                    a[r, sl] = a[r, sl] + b[r, sl]
                return carry
            lax.fori_loop(0, 16, row_body, 0)
            wd_[j] = pltpu.async_copy(a, out_hbm.at[pl.ds(lo + j * 16, 16)],
                                      wsem[j % 2])
        if nc >= 2:
            wd_[nc - 2].wait()
        wd_[nc - 1].wait()

    return combine


@jax.jit
def _moe(x, rw, wg, wu, wd):
    t, d = x.shape
    e = rw.shape[1]
    info = plsc.get_sparse_core_info()
    nw = info.num_cores * info.num_subcores

    nh = 2
    th = t // nh
    nblk = ((th * TOPK) // MB + e - 1 + 7) // 8 * 8
    s = nblk * MB

    dispatch = _make_dispatch(th, d, s, nw)
    combine = _make_combine(th, d, s, nw)

    meta = []
    for h in range(nh):
        xh = lax.slice_in_dim(x, h * th, (h + 1) * th, axis=0)
        d0, d1, w0, w1, eobp, src, nv = _router_call(xh, rw, nblk)
        meta.append((xh, d0.reshape(-1), d1.reshape(-1), w0.reshape(-1),
                     w1.reshape(-1), eobp.reshape(-1), src.reshape(-1),
                     nv.reshape(-1)))
    xsws = []
    for h in range(nh):
        xh, d0, d1, w0, w1, eobp, src, nv = meta[h]
        xsws.append(dispatch(d0, d1, w0, w1, xh))
    outs = []
    for h in range(nh):
        xh, d0, d1, w0, w1, eobp, src, nv = meta[h]
        xs, ws = xsws[h]
        ys = _gmm_call(eobp, src, nv, xs, ws.reshape(nblk, 1, MB),
                       wg, wu, wd)
        outs.append(combine(d0, d1, ys))
    return jnp.concatenate(outs, axis=0)


def kernel(hidden_states, router_w, w_gate, w_up, w_down):
    return _moe(hidden_states, router_w, w_gate, w_up, w_down)


# speedup vs baseline: 1.7406x; 1.7406x over previous
import functools

import jax
import jax.numpy as jnp
from jax import lax
from jax.experimental import pallas as pl
from jax.experimental.pallas import tpu as pltpu
from jax.experimental.pallas import tpu_sc as plsc

MB = 128
TOPK = 2


def _router_body(nblk, x_ref, rw_ref, d0_ref, d1_ref, w0_ref, w1_ref,
                 eob_ref, src_ref, nv_ref):
    xb = x_ref[...]
    t, _ = xb.shape
    e = rw_ref.shape[1]
    logits = jnp.dot(xb, rw_ref[...], preferred_element_type=jnp.float32)
    m = jnp.max(logits, axis=1, keepdims=True)
    p = jnp.exp(logits - m)
    p = p / jnp.sum(p, axis=1, keepdims=True)
    ii = lax.broadcasted_iota(jnp.int32, (t, e), 1)
    p1 = jnp.max(p, axis=1, keepdims=True)
    i1 = jnp.min(jnp.where(p == p1, ii, e), axis=1, keepdims=True)
    m1 = ii == i1
    pm = jnp.where(m1, -jnp.inf, p)
    p2 = jnp.max(pm, axis=1, keepdims=True)
    i2 = jnp.min(jnp.where(pm == p2, ii, e), axis=1, keepdims=True)
    m2 = ii == i2
    s = p1 + p2
    w0_ref[...] = (p1 / s).reshape(1, t)
    w1_ref[...] = (p2 / s).reshape(1, t)

    f1 = m1.astype(jnp.float32)
    f2 = m2.astype(jnp.float32)

    cb = 256
    ri = lax.broadcasted_iota(jnp.int32, (cb, cb), 0)
    ci = lax.broadcasted_iota(jnp.int32, (cb, cb), 1)
    tri = (ri >= ci).astype(jnp.float32)

    def cum(mat):
        chunks = []
        carry = jnp.zeros((1, e), jnp.float32)
        for c in range(t // cb):
            blk = mat[c * cb:(c + 1) * cb, :]
            cbk = jnp.dot(tri, blk, preferred_element_type=jnp.float32) + carry
            carry = cbk[cb - 1:cb, :]
            chunks.append(cbk)
        return jnp.concatenate(chunks, 0), carry

    c1, cnt1 = cum(f1)
    c2, cnt2 = cum(f2)
    cnt = cnt1 + cnt2
    nb = jnp.ceil(cnt * (1.0 / MB))
    eri = lax.broadcasted_iota(jnp.int32, (e, e), 0)
    eci = lax.broadcasted_iota(jnp.int32, (e, e), 1)
    triu = (eri < eci).astype(jnp.float32)
    pado = jnp.dot(nb, triu, preferred_element_type=jnp.float32) * MB
    pado_b = jnp.broadcast_to(pado, (t, e))
    cnt1_b = jnp.broadcast_to(cnt1, (t, e))
    d0 = jnp.sum(jnp.where(m1, pado_b + c1, 0.0), axis=1) - 1.0
    d1 = jnp.sum(jnp.where(m2, pado_b + cnt1_b + c2, 0.0), axis=1) - 1.0
    d0_ref[...] = d0.astype(jnp.int32).reshape(1, t)
    d1_ref[...] = d1.astype(jnp.int32).reshape(1, t)

    cume = jnp.dot(nb, triu,
                   preferred_element_type=jnp.float32).astype(jnp.int32)
    total = jnp.sum(nb).astype(jnp.int32)
    jb = lax.broadcasted_iota(jnp.int32, (nblk, e), 0)
    eob = jnp.sum((jb >= jnp.broadcast_to(cume, (nblk, e))).astype(
        jnp.int32), axis=1) - 1
    jv = lax.broadcasted_iota(jnp.int32, (nblk, 1), 0).reshape(nblk)
    lastv = jnp.maximum(total - 1, 0)
    srcv = jnp.where(jv < total, jv, lastv)
    jj = lax.broadcasted_iota(jnp.int32, (nblk, nblk), 1)
    sel = jj == srcv[:, None]
    eobp = jnp.sum(
        jnp.where(sel, jnp.broadcast_to(eob[None, :], (nblk, nblk)), 0),
        axis=1)
    eob_ref[...] = eobp.reshape(1, nblk)
    src_ref[...] = srcv.reshape(1, nblk)
    nv_ref[...] = total.reshape(1, 1)


@functools.partial(jax.jit, static_argnames=("nblk",))
def _router_call(x, rw, nblk):
    t, d = x.shape
    e = rw.shape[1]
    return pl.pallas_call(
        functools.partial(_router_body, nblk),
        grid=(1,),
        in_specs=[
            pl.BlockSpec((t, d), lambda i: (0, 0)),
            pl.BlockSpec((d, e), lambda i: (0, 0)),
        ],
        out_specs=[
            pl.BlockSpec((1, t), lambda i: (0, 0)),
            pl.BlockSpec((1, t), lambda i: (0, 0)),
            pl.BlockSpec((1, t), lambda i: (0, 0)),
            pl.BlockSpec((1, t), lambda i: (0, 0)),
            pl.BlockSpec((1, nblk), lambda i: (0, 0)),
            pl.BlockSpec((1, nblk), lambda i: (0, 0)),
            pl.BlockSpec((1, 1), lambda i: (0, 0)),
        ],
        out_shape=[
            jax.ShapeDtypeStruct((1, t), jnp.int32),
            jax.ShapeDtypeStruct((1, t), jnp.int32),
            jax.ShapeDtypeStruct((1, t), jnp.float32),
            jax.ShapeDtypeStruct((1, t), jnp.float32),
            jax.ShapeDtypeStruct((1, nblk), jnp.int32),
            jax.ShapeDtypeStruct((1, nblk), jnp.int32),
            jax.ShapeDtypeStruct((1, 1), jnp.int32),
        ],
    )(x, rw)


def _make_dispatch(t, d, s, nw):
    sw = s // nw
    ng = sw // 32
    mesh = plsc.VectorSubcoreMesh(core_axis_name="c", subcore_axis_name="s")

    @functools.partial(
        pl.kernel,
        mesh=mesh,
        compiler_params=pltpu.CompilerParams(needs_layout_passes=False),
        out_type=[
            jax.ShapeDtypeStruct((s,), jnp.int32),
            jax.ShapeDtypeStruct((s,), jnp.float32),
        ],
        scratch_types=[
            pltpu.VMEM((t,), jnp.int32),
            pltpu.VMEM((t,), jnp.int32),
            pltpu.VMEM((t,), jnp.float32),
            pltpu.VMEM((t,), jnp.float32),
            pltpu.VMEM((sw,), jnp.int32),
            pltpu.VMEM((sw,), jnp.float32),
        ],
    )
    def dispatch(d0_hbm, d1_hbm, w0_hbm, w1_hbm, it_hbm, ws_hbm,
                 d0_v, d1_v, w0_v, w1_v, inv1, wv):
        wid = lax.axis_index("s") * 2 + lax.axis_index("c")
        lo = wid * sw

        zi = jnp.zeros((16,), jnp.int32)
        zf = jnp.zeros((16,), jnp.float32)
        for c in range(sw // 16):
            inv1[pl.ds(c * 16, 16)] = zi
            wv[pl.ds(c * 16, 16)] = zf

        pltpu.sync_copy(d0_hbm, d0_v)
        pltpu.sync_copy(d1_hbm, d1_v)
        pltpu.sync_copy(w0_hbm, w0_v)
        pltpu.sync_copy(w1_hbm, w1_v)

        iota16 = lax.iota(jnp.int32, 16)

        def scan_pass(dv, wsrc):
            def body(c, carry):
                off = c * 16
                dd = dv[pl.ds(off, 16)]
                rel = dd - lo
                msk = (rel >= 0) & (rel < sw)
                relc = jnp.where(msk, rel, 0)
                toks = iota16 + off
                plsc.store_scatter(inv1, [relc], toks, mask=msk)
                plsc.store_scatter(wv, [relc], wsrc[pl.ds(off, 16)], mask=msk)
                return carry
            lax.fori_loop(0, t // 16, body, 0)

        scan_pass(d0_v, w0_v)
        scan_pass(d1_v, w1_v)

        pltpu.sync_copy(wv, ws_hbm.at[pl.ds(lo, sw)])
        pltpu.sync_copy(inv1, it_hbm.at[pl.ds(lo, sw)])

    return dispatch


def _gmm_body(eob_ref, src_ref, nv_ref, it_ref, x_ref, ws_ref, wg_ref,
              wu_ref, wd_ref, ys_ref, wg16, wu16, wd16, xb0, xb1, gsem):
    b = pl.program_id(0)
    nv = nv_ref[0]
    valid = b < nv
    mb, dd = xb0.shape
    changed = (b == 0) | (eob_ref[b] != eob_ref[jnp.maximum(b - 1, 0)])

    def fire(step, buf):
        blk = src_ref[step]

        def row(r, carry):
            tok = it_ref[blk * mb + r]
            pltpu.make_async_copy(
                x_ref.at[pl.ds(tok, 1), :], buf.at[pl.ds(r, 1), :], gsem
            ).start()
            return carry
        lax.fori_loop(0, mb, row, 0)

    @pl.when(b == 0)
    def _prime():
        fire(0, xb0)

    @pl.when(valid & changed)
    def _cast():
        wg16[...] = wg_ref[0].astype(jnp.bfloat16)
        wu16[...] = wu_ref[0].astype(jnp.bfloat16)
        wd16[...] = wd_ref[0].astype(jnp.bfloat16)

    def step(cur, nxt):
        pltpu.make_async_copy(x_ref.at[pl.ds(0, mb), :], cur, gsem).wait()

        @pl.when(b + 1 < nv)
        def _next():
            fire(b + 1, nxt)

        x16 = cur[...].astype(jnp.bfloat16)
        g = jnp.dot(x16, wg16[...], preferred_element_type=jnp.float32)
        u = jnp.dot(x16, wu16[...], preferred_element_type=jnp.float32)
        h = (g * jax.nn.sigmoid(g)) * u
        y = jnp.dot(h.astype(jnp.bfloat16), wd16[...],
                    preferred_element_type=jnp.float32)
        ys_ref[...] = y * ws_ref[0, 0, :][:, None]

    even = lax.rem(b, 2) == 0

    @pl.when(valid & even)
    def _even():
        step(xb0, xb1)

    @pl.when(valid & jnp.logical_not(even))
    def _odd():
        step(xb1, xb0)


@jax.jit
def _gmm_call(eobp, src, nv, invt, x, ws3, wg, wu, wd):
    t, d = x.shape
    e, _, f = wg.shape
    nblk = ws3.shape[0]
    s = nblk * MB
    grid_spec = pltpu.PrefetchScalarGridSpec(
        num_scalar_prefetch=4,
        grid=(nblk,),
        in_specs=[
            pl.BlockSpec(memory_space=pl.ANY),
            pl.BlockSpec((1, 1, MB),
                         lambda b, eo, sr, nv_, it: (sr[b], 0, 0)),
            pl.BlockSpec((1, d, f),
                         lambda b, eo, sr, nv_, it: (eo[b], 0, 0)),
            pl.BlockSpec((1, d, f),
                         lambda b, eo, sr, nv_, it: (eo[b], 0, 0)),
            pl.BlockSpec((1, f, d),
                         lambda b, eo, sr, nv_, it: (eo[b], 0, 0)),
        ],
        out_specs=pl.BlockSpec((MB, d), lambda b, eo, sr, nv_, it: (sr[b], 0)),
        scratch_shapes=[
            pltpu.VMEM((d, f), jnp.bfloat16),
            pltpu.VMEM((d, f), jnp.bfloat16),
            pltpu.VMEM((f, d), jnp.bfloat16),
            pltpu.VMEM((MB, d), jnp.float32),
            pltpu.VMEM((MB, d), jnp.float32),
            pltpu.SemaphoreType.DMA,
        ],
    )
    return pl.pallas_call(
        _gmm_body,
        grid_spec=grid_spec,
        out_shape=jax.ShapeDtypeStruct((s, d), jnp.float32),
    )(eobp, src, nv, invt, x, ws3, wg, wu, wd)


def _make_combine(t, d, s, nw):
    tw = t // nw
    nc = tw // 16
    mesh = plsc.VectorSubcoreMesh(core_axis_name="c", subcore_axis_name="s")

    @functools.partial(
        pl.kernel,
        mesh=mesh,
        compiler_params=pltpu.CompilerParams(needs_layout_passes=False),
        out_type=jax.ShapeDtypeStruct((t, d), jnp.float32),
        scratch_types=[
            pltpu.VMEM((tw,), jnp.int32),
            pltpu.VMEM((tw,), jnp.int32),
            pltpu.VMEM((16, d), jnp.float32),
            pltpu.VMEM((16, d), jnp.float32),
            pltpu.VMEM((16, d), jnp.float32),
            pltpu.VMEM((16, d), jnp.float32),
            pltpu.SemaphoreType.DMA,
            pltpu.SemaphoreType.DMA,
            pltpu.SemaphoreType.DMA,
            pltpu.SemaphoreType.DMA,
            pltpu.SemaphoreType.DMA,
            pltpu.SemaphoreType.DMA,
        ],
    )
    def combine(d0_hbm, d1_hbm, ys_hbm, out_hbm,
                d0t, d1t, a0, b0, a1, b1,
                sa0, sb0, sa1, sb1, sw0, sw1):
        wid = lax.axis_index("s") * 2 + lax.axis_index("c")
        lo = wid * tw
        pltpu.sync_copy(d0_hbm.at[pl.ds(lo, tw)], d0t)
        pltpu.sync_copy(d1_hbm.at[pl.ds(lo, tw)], d1t)

        abufs = [a0, a1]
        bbufs = [b0, b1]
        asem = [sa0, sa1]
        bsem = [sb0, sb1]
        wsem = [sw0, sw1]
        ga = [None] * nc
        gb = [None] * nc
        wd_ = [None] * nc

        def start(j):
            sl = pl.ds(j * 16, 16)
            ga[j] = pltpu.async_copy(ys_hbm.at[d0t.at[sl]], abufs[j % 2],
                                     asem[j % 2])
            gb[j] = pltpu.async_copy(ys_hbm.at[d1t.at[sl]], bbufs[j % 2],
                                     bsem[j % 2])

        start(0)
        for j in range(nc):
            ga[j].wait()
            gb[j].wait()
            if j + 1 < nc:
                if j - 1 >= 0:
                    wd_[j - 1].wait()
                start(j + 1)
            a = abufs[j % 2]
            b = bbufs[j % 2]

            def row_body(r, carry):
                for c in range(d // 16):
                    sl = pl.ds(c * 16, 16)
                    a[r, sl] = a[r, sl] + b[r, sl]
                return carry
            lax.fori_loop(0, 16, row_body, 0)
            wd_[j] = pltpu.async_copy(a, out_hbm.at[pl.ds(lo + j * 16, 16)],
                                      wsem[j % 2])
        if nc >= 2:
            wd_[nc - 2].wait()
        wd_[nc - 1].wait()

    return combine


@jax.jit
def _moe(x, rw, wg, wu, wd):
    t, d = x.shape
    e = rw.shape[1]
    nblk = ((t * TOPK) // MB + e - 1 + 7) // 8 * 8
    s = nblk * MB
    info = plsc.get_sparse_core_info()
    nw = info.num_cores * info.num_subcores

    d0, d1, w0, w1, eobp, src, nv = _router_call(x, rw, nblk)
    d0 = d0.reshape(-1)
    d1 = d1.reshape(-1)
    w0 = w0.reshape(-1)
    w1 = w1.reshape(-1)
    eobp = eobp.reshape(-1)
    src = src.reshape(-1)
    nv = nv.reshape(-1)

    invt, ws = _make_dispatch(t, d, s, nw)(d0, d1, w0, w1)
    ys = _gmm_call(eobp, src, nv, invt, x, ws.reshape(nblk, 1, MB),
                   wg, wu, wd)
    out = _make_combine(t, d, s, nw)(d0, d1, ys)
    return out


def kernel(hidden_states, router_w, w_gate, w_up, w_down):
    return _moe(hidden_states, router_w, w_gate, w_up, w_down)


# speedup vs baseline: 1.9431x; 1.1163x over previous
import functools

import jax
import jax.numpy as jnp
from jax import lax
from jax.experimental import pallas as pl
from jax.experimental.pallas import tpu as pltpu
from jax.experimental.pallas import tpu_sc as plsc

MB = 128
TOPK = 2


def _router_body(nblk, x_ref, rw_ref, d0_ref, d1_ref, w0_ref, w1_ref,
                 eob_ref, src_ref, nv_ref):
    xb = x_ref[...]
    t, _ = xb.shape
    e = rw_ref.shape[1]
    logits = jnp.dot(xb, rw_ref[...], preferred_element_type=jnp.float32)
    m = jnp.max(logits, axis=1, keepdims=True)
    p = jnp.exp(logits - m)
    p = p / jnp.sum(p, axis=1, keepdims=True)
    ii = lax.broadcasted_iota(jnp.int32, (t, e), 1)
    p1 = jnp.max(p, axis=1, keepdims=True)
    i1 = jnp.min(jnp.where(p == p1, ii, e), axis=1, keepdims=True)
    m1 = ii == i1
    pm = jnp.where(m1, -jnp.inf, p)
    p2 = jnp.max(pm, axis=1, keepdims=True)
    i2 = jnp.min(jnp.where(pm == p2, ii, e), axis=1, keepdims=True)
    m2 = ii == i2
    s = p1 + p2
    w0_ref[...] = (p1 / s).reshape(1, t)
    w1_ref[...] = (p2 / s).reshape(1, t)

    f1 = m1.astype(jnp.float32)
    f2 = m2.astype(jnp.float32)

    cb = 256
    ri = lax.broadcasted_iota(jnp.int32, (cb, cb), 0)
    ci = lax.broadcasted_iota(jnp.int32, (cb, cb), 1)
    tri = (ri >= ci).astype(jnp.float32)

    def cum(mat):
        chunks = []
        carry = jnp.zeros((1, e), jnp.float32)
        for c in range(t // cb):
            blk = mat[c * cb:(c + 1) * cb, :]
            cbk = jnp.dot(tri, blk, preferred_element_type=jnp.float32) + carry
            carry = cbk[cb - 1:cb, :]
            chunks.append(cbk)
        return jnp.concatenate(chunks, 0), carry

    c1, cnt1 = cum(f1)
    c2, cnt2 = cum(f2)
    cnt = cnt1 + cnt2
    nb = jnp.ceil(cnt * (1.0 / MB))
    eri = lax.broadcasted_iota(jnp.int32, (e, e), 0)
    eci = lax.broadcasted_iota(jnp.int32, (e, e), 1)
    triu = (eri < eci).astype(jnp.float32)
    pado = jnp.dot(nb, triu, preferred_element_type=jnp.float32) * MB
    pado_b = jnp.broadcast_to(pado, (t, e))
    cnt1_b = jnp.broadcast_to(cnt1, (t, e))
    d0 = jnp.sum(jnp.where(m1, pado_b + c1, 0.0), axis=1) - 1.0
    d1 = jnp.sum(jnp.where(m2, pado_b + cnt1_b + c2, 0.0), axis=1) - 1.0
    d0_ref[...] = d0.astype(jnp.int32).reshape(1, t)
    d1_ref[...] = d1.astype(jnp.int32).reshape(1, t)

    cume = jnp.dot(nb, triu,
                   preferred_element_type=jnp.float32).astype(jnp.int32)
    total = jnp.sum(nb).astype(jnp.int32)
    jb = lax.broadcasted_iota(jnp.int32, (nblk, e), 0)
    eob = jnp.sum((jb >= jnp.broadcast_to(cume, (nblk, e))).astype(
        jnp.int32), axis=1) - 1
    jv = lax.broadcasted_iota(jnp.int32, (nblk, 1), 0).reshape(nblk)
    lastv = jnp.maximum(total - 1, 0)
    srcv = jnp.where(jv < total, jv, lastv)
    jj = lax.broadcasted_iota(jnp.int32, (nblk, nblk), 1)
    sel = jj == srcv[:, None]
    eobp = jnp.sum(
        jnp.where(sel, jnp.broadcast_to(eob[None, :], (nblk, nblk)), 0),
        axis=1)
    eob_ref[...] = eobp.reshape(1, nblk)
    src_ref[...] = srcv.reshape(1, nblk)
    nv_ref[...] = total.reshape(1, 1)


@functools.partial(jax.jit, static_argnames=("nblk",))
def _router_call(x, rw, nblk):
    t, d = x.shape
    e = rw.shape[1]
    return pl.pallas_call(
        functools.partial(_router_body, nblk),
        grid=(1,),
        in_specs=[
            pl.BlockSpec((t, d), lambda i: (0, 0)),
            pl.BlockSpec((d, e), lambda i: (0, 0)),
        ],
        out_specs=[
            pl.BlockSpec((1, t), lambda i: (0, 0)),
            pl.BlockSpec((1, t), lambda i: (0, 0)),
            pl.BlockSpec((1, t), lambda i: (0, 0)),
            pl.BlockSpec((1, t), lambda i: (0, 0)),
            pl.BlockSpec((1, nblk), lambda i: (0, 0)),
            pl.BlockSpec((1, nblk), lambda i: (0, 0)),
            pl.BlockSpec((1, 1), lambda i: (0, 0)),
        ],
        out_shape=[
            jax.ShapeDtypeStruct((1, t), jnp.int32),
            jax.ShapeDtypeStruct((1, t), jnp.int32),
            jax.ShapeDtypeStruct((1, t), jnp.float32),
            jax.ShapeDtypeStruct((1, t), jnp.float32),
            jax.ShapeDtypeStruct((1, nblk), jnp.int32),
            jax.ShapeDtypeStruct((1, nblk), jnp.int32),
            jax.ShapeDtypeStruct((1, 1), jnp.int32),
        ],
    )(x, rw)


def _make_dispatch(t, d, s, nw):
    sw = s // nw
    ng = sw // 32
    mesh = plsc.VectorSubcoreMesh(core_axis_name="c", subcore_axis_name="s")

    @functools.partial(
        pl.kernel,
        mesh=mesh,
        compiler_params=pltpu.CompilerParams(needs_layout_passes=False),
        out_type=[
            jax.ShapeDtypeStruct((s,), jnp.int32),
            jax.ShapeDtypeStruct((s,), jnp.float32),
        ],
        scratch_types=[
            pltpu.VMEM((t,), jnp.int32),
            pltpu.VMEM((t,), jnp.int32),
            pltpu.VMEM((t,), jnp.float32),
            pltpu.VMEM((t,), jnp.float32),
            pltpu.VMEM((sw,), jnp.int32),
            pltpu.VMEM((sw,), jnp.float32),
        ],
    )
    def dispatch(d0_hbm, d1_hbm, w0_hbm, w1_hbm, it_hbm, ws_hbm,
                 d0_v, d1_v, w0_v, w1_v, inv1, wv):
        wid = lax.axis_index("s") * 2 + lax.axis_index("c")
        lo = wid * sw

        zi = jnp.zeros((16,), jnp.int32)
        zf = jnp.zeros((16,), jnp.float32)
        for c in range(sw // 16):
            inv1[pl.ds(c * 16, 16)] = zi
            wv[pl.ds(c * 16, 16)] = zf

        pltpu.sync_copy(d0_hbm, d0_v)
        pltpu.sync_copy(d1_hbm, d1_v)
        pltpu.sync_copy(w0_hbm, w0_v)
        pltpu.sync_copy(w1_hbm, w1_v)

        iota16 = lax.iota(jnp.int32, 16)

        def scan_pass(dv, wsrc):
            def body(c, carry):
                off = c * 16
                dd = dv[pl.ds(off, 16)]
                rel = dd - lo
                msk = (rel >= 0) & (rel < sw)
                relc = jnp.where(msk, rel, 0)
                toks = iota16 + off
                plsc.store_scatter(inv1, [relc], toks, mask=msk)
                plsc.store_scatter(wv, [relc], wsrc[pl.ds(off, 16)], mask=msk)
                return carry
            lax.fori_loop(0, t // 16, body, 0)

        scan_pass(d0_v, w0_v)
        scan_pass(d1_v, w1_v)

        pltpu.sync_copy(wv, ws_hbm.at[pl.ds(lo, sw)])
        pltpu.sync_copy(inv1, it_hbm.at[pl.ds(lo, sw)])

    return dispatch


def _gmm_body(eob_ref, src_ref, nv_ref, it_ref, x_ref, ws_ref, wg_ref,
              wu_ref, wd_ref, ys_ref, wg16, wu16, wd16, xb0, xb1, gsem):
    b = pl.program_id(0)
    nv = nv_ref[0]
    valid = b < nv
    mb, dd = xb0.shape
    changed = (b == 0) | (eob_ref[b] != eob_ref[jnp.maximum(b - 1, 0)])

    def fire(step, buf):
        blk = src_ref[step]
        base = blk * mb
        for r in range(mb):
            tok = it_ref[base + r]
            pltpu.make_async_copy(
                x_ref.at[pl.ds(tok, 1), :], buf.at[pl.ds(r, 1), :], gsem
            ).start()

    @pl.when(b == 0)
    def _prime():
        fire(0, xb0)

    @pl.when(valid & changed)
    def _cast():
        wg16[...] = wg_ref[0].astype(jnp.bfloat16)
        wu16[...] = wu_ref[0].astype(jnp.bfloat16)
        wd16[...] = wd_ref[0].astype(jnp.bfloat16)

    def step(cur, nxt):
        pltpu.make_async_copy(x_ref.at[pl.ds(0, mb), :], cur, gsem).wait()

        @pl.when(b + 1 < nv)
        def _next():
            fire(b + 1, nxt)

        x16 = cur[...].astype(jnp.bfloat16)
        g = jnp.dot(x16, wg16[...], preferred_element_type=jnp.float32)
        u = jnp.dot(x16, wu16[...], preferred_element_type=jnp.float32)
        h = (g * jax.nn.sigmoid(g)) * u
        y = jnp.dot(h.astype(jnp.bfloat16), wd16[...],
                    preferred_element_type=jnp.float32)
        ys_ref[...] = y * ws_ref[0, 0, :][:, None]

    even = lax.rem(b, 2) == 0

    @pl.when(valid & even)
    def _even():
        step(xb0, xb1)

    @pl.when(valid & jnp.logical_not(even))
    def _odd():
        step(xb1, xb0)


@jax.jit
def _gmm_call(eobp, src, nv, invt, x, ws3, wg, wu, wd):
    t, d = x.shape
    e, _, f = wg.shape
    nblk = ws3.shape[0]
    s = nblk * MB
    grid_spec = pltpu.PrefetchScalarGridSpec(
        num_scalar_prefetch=4,
        grid=(nblk,),
        in_specs=[
            pl.BlockSpec(memory_space=pl.ANY),
            pl.BlockSpec((1, 1, MB),
                         lambda b, eo, sr, nv_, it: (sr[b], 0, 0)),
            pl.BlockSpec((1, d, f),
                         lambda b, eo, sr, nv_, it: (eo[b], 0, 0)),
            pl.BlockSpec((1, d, f),
                         lambda b, eo, sr, nv_, it: (eo[b], 0, 0)),
            pl.BlockSpec((1, f, d),
                         lambda b, eo, sr, nv_, it: (eo[b], 0, 0)),
        ],
        out_specs=pl.BlockSpec((MB, d), lambda b, eo, sr, nv_, it: (sr[b], 0)),
        scratch_shapes=[
            pltpu.VMEM((d, f), jnp.bfloat16),
            pltpu.VMEM((d, f), jnp.bfloat16),
            pltpu.VMEM((f, d), jnp.bfloat16),
            pltpu.VMEM((MB, d), jnp.float32),
            pltpu.VMEM((MB, d), jnp.float32),
            pltpu.SemaphoreType.DMA,
        ],
    )
    return pl.pallas_call(
        _gmm_body,
        grid_spec=grid_spec,
        out_shape=jax.ShapeDtypeStruct((s, d), jnp.float32),
    )(eobp, src, nv, invt, x, ws3, wg, wu, wd)


def _make_combine(t, d, s, nw):
    tw = t // nw
    nc = tw // 16
    mesh = plsc.VectorSubcoreMesh(core_axis_name="c", subcore_axis_name="s")

    @functools.partial(
        pl.kernel,
        mesh=mesh,
        compiler_params=pltpu.CompilerParams(needs_layout_passes=False),
        out_type=jax.ShapeDtypeStruct((t, d), jnp.float32),
        scratch_types=[
            pltpu.VMEM((tw,), jnp.int32),
            pltpu.VMEM((tw,), jnp.int32),
            pltpu.VMEM((16, d), jnp.float32),
            pltpu.VMEM((16, d), jnp.float32),
            pltpu.VMEM((16, d), jnp.float32),
            pltpu.VMEM((16, d), jnp.float32),
            pltpu.SemaphoreType.DMA,
            pltpu.SemaphoreType.DMA,
            pltpu.SemaphoreType.DMA,
            pltpu.SemaphoreType.DMA,
            pltpu.SemaphoreType.DMA,
            pltpu.SemaphoreType.DMA,
        ],
    )
    def combine(d0_hbm, d1_hbm, ys_hbm, out_hbm,
                d0t, d1t, a0, b0, a1, b1,
                sa0, sb0, sa1, sb1, sw0, sw1):
        wid = lax.axis_index("s") * 2 + lax.axis_index("c")
        lo = wid * tw
        pltpu.sync_copy(d0_hbm.at[pl.ds(lo, tw)], d0t)
        pltpu.sync_copy(d1_hbm.at[pl.ds(lo, tw)], d1t)

        abufs = [a0, a1]
        bbufs = [b0, b1]
        asem = [sa0, sa1]
        bsem = [sb0, sb1]
        wsem = [sw0, sw1]
        ga = [None] * nc
        gb = [None] * nc
        wd_ = [None] * nc

        def start(j):
            sl = pl.ds(j * 16, 16)
            ga[j] = pltpu.async_copy(ys_hbm.at[d0t.at[sl]], abufs[j % 2],
                                     asem[j % 2])
            gb[j] = pltpu.async_copy(ys_hbm.at[d1t.at[sl]], bbufs[j % 2],
                                     bsem[j % 2])

        start(0)
        for j in range(nc):
            ga[j].wait()
            gb[j].wait()
            if j + 1 < nc:
                if j - 1 >= 0:
                    wd_[j - 1].wait()
                start(j + 1)
            a = abufs[j % 2]
            b = bbufs[j % 2]

            def row_body(r, carry):
                for c in range(d // 16):
                    sl = pl.ds(c * 16, 16)
                    a[r, sl] = a[r, sl] + b[r, sl]
                return carry
            lax.fori_loop(0, 16, row_body, 0)
            wd_[j] = pltpu.async_copy(a, out_hbm.at[pl.ds(lo + j * 16, 16)],
                                      wsem[j % 2])
        if nc >= 2:
            wd_[nc - 2].wait()
        wd_[nc - 1].wait()

    return combine


@jax.jit
def _moe(x, rw, wg, wu, wd):
    t, d = x.shape
    e = rw.shape[1]
    nblk = ((t * TOPK) // MB + e - 1 + 7) // 8 * 8
    s = nblk * MB
    info = plsc.get_sparse_core_info()
    nw = info.num_cores * info.num_subcores

    d0, d1, w0, w1, eobp, src, nv = _router_call(x, rw, nblk)
    d0 = d0.reshape(-1)
    d1 = d1.reshape(-1)
    w0 = w0.reshape(-1)
    w1 = w1.reshape(-1)
    eobp = eobp.reshape(-1)
    src = src.reshape(-1)
    nv = nv.reshape(-1)

    invt, ws = _make_dispatch(t, d, s, nw)(d0, d1, w0, w1)
    ys = _gmm_call(eobp, src, nv, invt, x, ws.reshape(nblk, 1, MB),
                   wg, wu, wd)
    out = _make_combine(t, d, s, nw)(d0, d1, ys)
    return out


def kernel(hidden_states, router_w, w_gate, w_up, w_down):
    return _moe(hidden_states, router_w, w_gate, w_up, w_down)


# speedup vs baseline: 1.9596x; 1.0085x over previous
import functools

import jax
import jax.numpy as jnp
from jax import lax
from jax.experimental import pallas as pl
from jax.experimental.pallas import tpu as pltpu
from jax.experimental.pallas import tpu_sc as plsc

MB = 128
TOPK = 2


def _router_body(nblk, x_ref, rw_ref, d0_ref, d1_ref, w0_ref, w1_ref,
                 eob_ref, src_ref, nv_ref):
    xb = x_ref[...]
    t, _ = xb.shape
    e = rw_ref.shape[1]
    logits = jnp.dot(xb, rw_ref[...], preferred_element_type=jnp.float32)
    m = jnp.max(logits, axis=1, keepdims=True)
    p = jnp.exp(logits - m)
    p = p / jnp.sum(p, axis=1, keepdims=True)
    ii = lax.broadcasted_iota(jnp.int32, (t, e), 1)
    p1 = jnp.max(p, axis=1, keepdims=True)
    i1 = jnp.min(jnp.where(p == p1, ii, e), axis=1, keepdims=True)
    m1 = ii == i1
    pm = jnp.where(m1, -jnp.inf, p)
    p2 = jnp.max(pm, axis=1, keepdims=True)
    i2 = jnp.min(jnp.where(pm == p2, ii, e), axis=1, keepdims=True)
    m2 = ii == i2
    s = p1 + p2
    w0_ref[...] = (p1 / s).reshape(1, t)
    w1_ref[...] = (p2 / s).reshape(1, t)

    f1 = m1.astype(jnp.float32)
    f2 = m2.astype(jnp.float32)

    cb = 256
    ri = lax.broadcasted_iota(jnp.int32, (cb, cb), 0)
    ci = lax.broadcasted_iota(jnp.int32, (cb, cb), 1)
    tri = (ri >= ci).astype(jnp.float32)

    def cum(mat):
        chunks = []
        carry = jnp.zeros((1, e), jnp.float32)
        for c in range(t // cb):
            blk = mat[c * cb:(c + 1) * cb, :]
            cbk = jnp.dot(tri, blk, preferred_element_type=jnp.float32) + carry
            carry = cbk[cb - 1:cb, :]
            chunks.append(cbk)
        return jnp.concatenate(chunks, 0), carry

    c1, cnt1 = cum(f1)
    c2, cnt2 = cum(f2)
    cnt = cnt1 + cnt2
    nb = jnp.ceil(cnt * (1.0 / MB))
    eri = lax.broadcasted_iota(jnp.int32, (e, e), 0)
    eci = lax.broadcasted_iota(jnp.int32, (e, e), 1)
    triu = (eri < eci).astype(jnp.float32)
    pado = jnp.dot(nb, triu, preferred_element_type=jnp.float32) * MB
    pado_b = jnp.broadcast_to(pado, (t, e))
    cnt1_b = jnp.broadcast_to(cnt1, (t, e))
    d0 = jnp.sum(jnp.where(m1, pado_b + c1, 0.0), axis=1) - 1.0
    d1 = jnp.sum(jnp.where(m2, pado_b + cnt1_b + c2, 0.0), axis=1) - 1.0
    d0_ref[...] = d0.astype(jnp.int32).reshape(1, t)
    d1_ref[...] = d1.astype(jnp.int32).reshape(1, t)

    cume = jnp.dot(nb, triu,
                   preferred_element_type=jnp.float32).astype(jnp.int32)
    total = jnp.sum(nb).astype(jnp.int32)
    jb = lax.broadcasted_iota(jnp.int32, (nblk, e), 0)
    eob = jnp.sum((jb >= jnp.broadcast_to(cume, (nblk, e))).astype(
        jnp.int32), axis=1) - 1
    jv = lax.broadcasted_iota(jnp.int32, (nblk, 1), 0).reshape(nblk)
    lastv = jnp.maximum(total - 1, 0)
    srcv = jnp.where(jv < total, jv, lastv)
    jj = lax.broadcasted_iota(jnp.int32, (nblk, nblk), 1)
    sel = jj == srcv[:, None]
    eobp = jnp.sum(
        jnp.where(sel, jnp.broadcast_to(eob[None, :], (nblk, nblk)), 0),
        axis=1)
    eob_ref[...] = eobp.reshape(1, nblk)
    src_ref[...] = srcv.reshape(1, nblk)
    nv_ref[...] = total.reshape(1, 1)


@functools.partial(jax.jit, static_argnames=("nblk",))
def _router_call(x, rw, nblk):
    t, d = x.shape
    e = rw.shape[1]
    return pl.pallas_call(
        functools.partial(_router_body, nblk),
        grid=(1,),
        in_specs=[
            pl.BlockSpec((t, d), lambda i: (0, 0)),
            pl.BlockSpec((d, e), lambda i: (0, 0)),
        ],
        out_specs=[
            pl.BlockSpec((1, t), lambda i: (0, 0)),
            pl.BlockSpec((1, t), lambda i: (0, 0)),
            pl.BlockSpec((1, t), lambda i: (0, 0)),
            pl.BlockSpec((1, t), lambda i: (0, 0)),
            pl.BlockSpec((1, nblk), lambda i: (0, 0)),
            pl.BlockSpec((1, nblk), lambda i: (0, 0)),
            pl.BlockSpec((1, 1), lambda i: (0, 0)),
        ],
        out_shape=[
            jax.ShapeDtypeStruct((1, t), jnp.int32),
            jax.ShapeDtypeStruct((1, t), jnp.int32),
            jax.ShapeDtypeStruct((1, t), jnp.float32),
            jax.ShapeDtypeStruct((1, t), jnp.float32),
            jax.ShapeDtypeStruct((1, nblk), jnp.int32),
            jax.ShapeDtypeStruct((1, nblk), jnp.int32),
            jax.ShapeDtypeStruct((1, 1), jnp.int32),
        ],
    )(x, rw)


def _make_dispatch(t, d, s, nw):
    sw = s // nw
    ng = sw // 32
    mesh = plsc.VectorSubcoreMesh(core_axis_name="c", subcore_axis_name="s")

    @functools.partial(
        pl.kernel,
        mesh=mesh,
        compiler_params=pltpu.CompilerParams(needs_layout_passes=False),
        out_type=[
            jax.ShapeDtypeStruct((s,), jnp.int32),
            jax.ShapeDtypeStruct((s,), jnp.float32),
        ],
        scratch_types=[
            pltpu.VMEM((t,), jnp.int32),
            pltpu.VMEM((t,), jnp.int32),
            pltpu.VMEM((t,), jnp.float32),
            pltpu.VMEM((t,), jnp.float32),
            pltpu.VMEM((sw,), jnp.int32),
            pltpu.VMEM((sw,), jnp.float32),
        ],
    )
    def dispatch(d0_hbm, d1_hbm, w0_hbm, w1_hbm, it_hbm, ws_hbm,
                 d0_v, d1_v, w0_v, w1_v, inv1, wv):
        wid = lax.axis_index("s") * 2 + lax.axis_index("c")
        lo = wid * sw

        zi = jnp.zeros((16,), jnp.int32)
        zf = jnp.zeros((16,), jnp.float32)
        for c in range(sw // 16):
            inv1[pl.ds(c * 16, 16)] = zi
            wv[pl.ds(c * 16, 16)] = zf

        pltpu.sync_copy(d0_hbm, d0_v)
        pltpu.sync_copy(d1_hbm, d1_v)
        pltpu.sync_copy(w0_hbm, w0_v)
        pltpu.sync_copy(w1_hbm, w1_v)

        iota16 = lax.iota(jnp.int32, 16)

        def scan_pass(dv, wsrc):
            def body(c, carry):
                off = c * 16
                dd = dv[pl.ds(off, 16)]
                rel = dd - lo
                msk = (rel >= 0) & (rel < sw)
                relc = jnp.where(msk, rel, 0)
                toks = iota16 + off
                plsc.store_scatter(inv1, [relc], toks, mask=msk)
                plsc.store_scatter(wv, [relc], wsrc[pl.ds(off, 16)], mask=msk)
                return carry
            lax.fori_loop(0, t // 16, body, 0)

        scan_pass(d0_v, w0_v)
        scan_pass(d1_v, w1_v)

        pltpu.sync_copy(wv, ws_hbm.at[pl.ds(lo, sw)])
        pltpu.sync_copy(inv1, it_hbm.at[pl.ds(lo, sw)])

    return dispatch


def _gmm_body(eob_ref, src_ref, nv_ref, it_ref, x_ref, ws_ref, wg_ref,
              wu_ref, wd_ref, ys_ref, wg16, wu16, wd16, xb0, xb1,
              gsem0, gsem1):
    b = pl.program_id(0)
    nv = nv_ref[0]
    valid = b < nv
    mb, dd = xb0.shape
    changed = (b == 0) | (eob_ref[b] != eob_ref[jnp.maximum(b - 1, 0)])

    def fire(step, buf, sem):
        blk = src_ref[step]
        base = blk * mb
        for r in range(mb):
            tok = it_ref[base + r]
            pltpu.make_async_copy(
                x_ref.at[pl.ds(tok, 1), :], buf.at[pl.ds(r, 1), :], sem
            ).start()

    @pl.when(b == 0)
    def _prime():
        fire(0, xb0, gsem0)

    @pl.when(valid & changed)
    def _cast():
        wg16[...] = wg_ref[0].astype(jnp.bfloat16)
        wu16[...] = wu_ref[0].astype(jnp.bfloat16)
        wd16[...] = wd_ref[0].astype(jnp.bfloat16)

    def step(cur, nxt, csem, nsem):
        @pl.when(b + 1 < nv)
        def _next():
            fire(b + 1, nxt, nsem)

        pltpu.make_async_copy(x_ref.at[pl.ds(0, mb), :], cur, csem).wait()

        x16 = cur[...].astype(jnp.bfloat16)
        g = jnp.dot(x16, wg16[...], preferred_element_type=jnp.float32)
        u = jnp.dot(x16, wu16[...], preferred_element_type=jnp.float32)
        h = (g * jax.nn.sigmoid(g)) * u
        y = jnp.dot(h.astype(jnp.bfloat16), wd16[...],
                    preferred_element_type=jnp.float32)
        ys_ref[...] = y * ws_ref[0, 0, :][:, None]

    even = lax.rem(b, 2) == 0

    @pl.when(valid & even)
    def _even():
        step(xb0, xb1, gsem0, gsem1)

    @pl.when(valid & jnp.logical_not(even))
    def _odd():
        step(xb1, xb0, gsem1, gsem0)


@jax.jit
def _gmm_call(eobp, src, nv, invt, x, ws3, wg, wu, wd):
    t, d = x.shape
    e, _, f = wg.shape
    nblk = ws3.shape[0]
    s = nblk * MB
    grid_spec = pltpu.PrefetchScalarGridSpec(
        num_scalar_prefetch=4,
        grid=(nblk,),
        in_specs=[
            pl.BlockSpec(memory_space=pl.ANY),
            pl.BlockSpec((1, 1, MB),
                         lambda b, eo, sr, nv_, it: (sr[b], 0, 0)),
            pl.BlockSpec((1, d, f),
                         lambda b, eo, sr, nv_, it: (eo[b], 0, 0)),
            pl.BlockSpec((1, d, f),
                         lambda b, eo, sr, nv_, it: (eo[b], 0, 0)),
            pl.BlockSpec((1, f, d),
                         lambda b, eo, sr, nv_, it: (eo[b], 0, 0)),
        ],
        out_specs=pl.BlockSpec((MB, d), lambda b, eo, sr, nv_, it: (sr[b], 0)),
        scratch_shapes=[
            pltpu.VMEM((d, f), jnp.bfloat16),
            pltpu.VMEM((d, f), jnp.bfloat16),
            pltpu.VMEM((f, d), jnp.bfloat16),
            pltpu.VMEM((MB, d), jnp.float32),
            pltpu.VMEM((MB, d), jnp.float32),
            pltpu.SemaphoreType.DMA,
            pltpu.SemaphoreType.DMA,
        ],
    )
    return pl.pallas_call(
        _gmm_body,
        grid_spec=grid_spec,
        out_shape=jax.ShapeDtypeStruct((s, d), jnp.float32),
    )(eobp, src, nv, invt, x, ws3, wg, wu, wd)


def _make_combine(t, d, s, nw):
    tw = t // nw
    nc = tw // 16
    mesh = plsc.VectorSubcoreMesh(core_axis_name="c", subcore_axis_name="s")

    @functools.partial(
        pl.kernel,
        mesh=mesh,
        compiler_params=pltpu.CompilerParams(needs_layout_passes=False),
        out_type=jax.ShapeDtypeStruct((t, d), jnp.float32),
        scratch_types=[
            pltpu.VMEM((tw,), jnp.int32),
            pltpu.VMEM((tw,), jnp.int32),
            pltpu.VMEM((16, d), jnp.float32),
            pltpu.VMEM((16, d), jnp.float32),
            pltpu.VMEM((16, d), jnp.float32),
            pltpu.VMEM((16, d), jnp.float32),
            pltpu.SemaphoreType.DMA,
            pltpu.SemaphoreType.DMA,
            pltpu.SemaphoreType.DMA,
            pltpu.SemaphoreType.DMA,
            pltpu.SemaphoreType.DMA,
            pltpu.SemaphoreType.DMA,
        ],
    )
    def combine(d0_hbm, d1_hbm, ys_hbm, out_hbm,
                d0t, d1t, a0, b0, a1, b1,
                sa0, sb0, sa1, sb1, sw0, sw1):
        wid = lax.axis_index("s") * 2 + lax.axis_index("c")
        lo = wid * tw
        pltpu.sync_copy(d0_hbm.at[pl.ds(lo, tw)], d0t)
        pltpu.sync_copy(d1_hbm.at[pl.ds(lo, tw)], d1t)

        abufs = [a0, a1]
        bbufs = [b0, b1]
        asem = [sa0, sa1]
        bsem = [sb0, sb1]
        wsem = [sw0, sw1]
        ga = [None] * nc
        gb = [None] * nc
        wd_ = [None] * nc

        def start(j):
            sl = pl.ds(j * 16, 16)
            ga[j] = pltpu.async_copy(ys_hbm.at[d0t.at[sl]], abufs[j % 2],
                                     asem[j % 2])
            gb[j] = pltpu.async_copy(ys_hbm.at[d1t.at[sl]], bbufs[j % 2],
                                     bsem[j % 2])

        start(0)
        for j in range(nc):
            ga[j].wait()
            gb[j].wait()
            if j + 1 < nc:
                if j - 1 >= 0:
                    wd_[j - 1].wait()
                start(j + 1)
            a = abufs[j % 2]
            b = bbufs[j % 2]

            def row_body(r, carry):
                for c in range(d // 16):
                    sl = pl.ds(c * 16, 16)
                    a[r, sl] = a[r, sl] + b[r, sl]
                return carry
            lax.fori_loop(0, 16, row_body, 0)
            wd_[j] = pltpu.async_copy(a, out_hbm.at[pl.ds(lo + j * 16, 16)],
                                      wsem[j % 2])
        if nc >= 2:
            wd_[nc - 2].wait()
        wd_[nc - 1].wait()

    return combine


@jax.jit
def _moe(x, rw, wg, wu, wd):
    t, d = x.shape
    e = rw.shape[1]
    nblk = ((t * TOPK) // MB + e - 1 + 7) // 8 * 8
    s = nblk * MB
    info = plsc.get_sparse_core_info()
    nw = info.num_cores * info.num_subcores

    d0, d1, w0, w1, eobp, src, nv = _router_call(x, rw, nblk)
    d0 = d0.reshape(-1)
    d1 = d1.reshape(-1)
    w0 = w0.reshape(-1)
    w1 = w1.reshape(-1)
    eobp = eobp.reshape(-1)
    src = src.reshape(-1)
    nv = nv.reshape(-1)

    invt, ws = _make_dispatch(t, d, s, nw)(d0, d1, w0, w1)
    ys = _gmm_call(eobp, src, nv, invt, x, ws.reshape(nblk, 1, MB),
                   wg, wu, wd)
    out = _make_combine(t, d, s, nw)(d0, d1, ys)
    return out


def kernel(hidden_states, router_w, w_gate, w_up, w_down):
    return _moe(hidden_states, router_w, w_gate, w_up, w_down)
